# baseline (speedup 1.0000x reference)
import functools

import jax
import jax.numpy as jnp
from jax import lax
from jax.experimental import pallas as pl
from jax.experimental.pallas import tpu as pltpu
from jax.experimental.pallas import tpu_sc as plsc

NC = 2
NS = 16
NW = NC * NS

BATCH = 16384
FIELDS = 100
DIM = 32

BBLK = 128
BT_PER_TILE = BATCH // (NW * BBLK)
UNITS = BT_PER_TILE * FIELDS
DT = DIM // 8
NBUF = 8


def _extract_icol(xblk, icol, bt_local, f):
    iota = lax.iota(jnp.int32, 16)
    fvec = jnp.zeros((16,), jnp.int32) + f
    for bg in range(8):
        rows = iota + (bt_local * BBLK + bg * 16)
        vals = plsc.load_gather(xblk, [rows, fvec])
        icol[pl.ds(bg * 16, 16)] = vals


def _transpose_unit(rows, trans):
    iota = lax.iota(jnp.int32, 16)
    rvecs = [iota + bg * 16 for bg in range(8)]

    @plsc.parallel_loop(0, 16, unroll=2)
    def _(k):
        ddvec = jnp.remainder(iota + k, 16)
        drv = ddvec & 7
        dtv0 = ddvec >> 3
        for h in range(2):
            cvec = ddvec + h * 16
            dtv = dtv0 + h * 2
            for bg in range(8):
                vals = plsc.load_gather(rows, [rvecs[bg], cvec])
                plsc.store_scatter(trans, [dtv, drv, rvecs[bg]], vals)


@functools.lru_cache(maxsize=None)
def _build_gather():
    mesh = plsc.VectorSubcoreMesh(core_axis_name="c", subcore_axis_name="s")

    @functools.partial(
        pl.kernel,
        out_type=jax.ShapeDtypeStruct((FIELDS, DT, BATCH // BBLK, 8, BBLK),
                                      jnp.float32),
        mesh=mesh,
        compiler_params=pltpu.CompilerParams(use_tc_tiling_on_sc=False,
                                             needs_layout_passes=False,
                                             disable_bounds_checks=True),
        scratch_types=(
            [pltpu.VMEM((NW * BT_PER_TILE * BBLK // NW, FIELDS), jnp.int32)]
            + [pltpu.VMEM((BBLK,), jnp.int32) for _ in range(NBUF)]
            + [pltpu.VMEM((BBLK, DIM), jnp.float32) for _ in range(NBUF)]
            + [pltpu.VMEM((DT, 8, BBLK), jnp.float32) for _ in range(NBUF)]
            + [pltpu.SemaphoreType.DMA for _ in range(2 * NBUF)]
        ),
    )
    def gather(table_hbm, x_hbm, out_hbm, xblk, *bufs):
        icol = list(bufs[0:NBUF])
        rows = list(bufs[NBUF:2 * NBUF])
        trans = list(bufs[2 * NBUF:3 * NBUF])
        gsem = list(bufs[3 * NBUF:4 * NBUF])
        osem = list(bufs[4 * NBUF:5 * NBUF])

        wid = lax.axis_index("s") * NC + lax.axis_index("c")
        b_per_tile = BT_PER_TILE * BBLK
        bbase = wid * b_per_tile

        pltpu.sync_copy(x_hbm.at[pl.ds(bbase, b_per_tile)], xblk)

        def unit_btf(u):
            bt_local = u // FIELDS
            f = u - bt_local * FIELDS
            return bt_local, f

        def start_gather(u, p):
            bt_local, f = unit_btf(u)
            _extract_icol(xblk, icol[p], bt_local, f)
            pltpu.async_copy(table_hbm.at[icol[p]], rows[p], gsem[p])

        def finish_unit(u, p, drain_prev, start_next):
            bt_local, f = unit_btf(u)
            btg = wid * BT_PER_TILE + bt_local
            pltpu.make_async_copy(table_hbm.at[icol[p]], rows[p],
                                  gsem[p]).wait()
            if drain_prev:
                pltpu.make_async_copy(trans[p], out_hbm.at[f, :, btg],
                                      osem[p]).wait()
            _transpose_unit(rows[p], trans[p])
            pltpu.async_copy(trans[p], out_hbm.at[f, :, btg], osem[p])
            if start_next:
                start_gather(u + NBUF, p)

        for p in range(NBUF):
            start_gather(p, p)
        for p in range(NBUF):
            finish_unit(p, p, drain_prev=False, start_next=True)

        @pl.loop(NBUF, UNITS - NBUF, step=NBUF)
        def _(u):
            for p in range(NBUF):
                finish_unit(u + p, p, drain_prev=True, start_next=True)

        for p in range(NBUF):
            finish_unit(UNITS - NBUF + p, p, drain_prev=True,
                        start_next=False)

        last_bt = BT_PER_TILE - 1
        btg = wid * BT_PER_TILE + last_bt
        for p in range(NBUF):
            pltpu.make_async_copy(trans[p],
                                  out_hbm.at[FIELDS - NBUF + p, :, btg],
                                  osem[p]).wait()

    return gather


def kernel(x, table):
    out5 = _build_gather()(table, x)
    out = out5.transpose(2, 4, 0, 1, 3).reshape(BATCH, FIELDS, DIM)
    return out


# speedup vs baseline: 1.0806x; 1.0806x over previous
import functools

import jax
import jax.numpy as jnp
from jax import lax
from jax.experimental import pallas as pl
from jax.experimental.pallas import tpu as pltpu
from jax.experimental.pallas import tpu_sc as plsc

NC = 2
NS = 16
NW = NC * NS

BATCH = 16384
FIELDS = 100
DIM = 32

BBLK = 128
BT_PER_TILE = BATCH // (NW * BBLK)
UNITS = BT_PER_TILE * FIELDS
DT = DIM // 8
NBUF = 5


def _extract_icol(xblk, icol, bt_local, f):
    iota = lax.iota(jnp.int32, 16)
    fvec = jnp.zeros((16,), jnp.int32) + f
    for bg in range(8):
        rows = iota + (bt_local * BBLK + bg * 16)
        vals = plsc.load_gather(xblk, [rows, fvec])
        icol[pl.ds(bg * 16, 16)] = vals


def _transpose_unit(rows, trans):
    iota = lax.iota(jnp.int32, 16)
    rvecs = [iota + bg * 16 for bg in range(8)]

    @plsc.parallel_loop(0, 16, unroll=2)
    def _(k):
        ddvec = jnp.remainder(iota + k, 16)
        drv = ddvec & 7
        dtv0 = ddvec >> 3
        for h in range(2):
            cvec = ddvec + h * 16
            dtv = dtv0 + h * 2
            for bg in range(8):
                vals = plsc.load_gather(rows, [rvecs[bg], cvec])
                plsc.store_scatter(trans, [dtv, drv, rvecs[bg]], vals)


@functools.lru_cache(maxsize=None)
def _build_gather():
    mesh = plsc.VectorSubcoreMesh(core_axis_name="c", subcore_axis_name="s")

    @functools.partial(
        pl.kernel,
        out_type=jax.ShapeDtypeStruct((FIELDS, DT, BATCH // BBLK, 8, BBLK),
                                      jnp.float32),
        mesh=mesh,
        compiler_params=pltpu.CompilerParams(use_tc_tiling_on_sc=False,
                                             needs_layout_passes=False,
                                             disable_bounds_checks=True),
        scratch_types=(
            [pltpu.VMEM((NW * BT_PER_TILE * BBLK // NW, FIELDS), jnp.int32)]
            + [pltpu.VMEM((BBLK,), jnp.int32) for _ in range(NBUF)]
            + [pltpu.VMEM((BBLK, DIM), jnp.float32) for _ in range(NBUF)]
            + [pltpu.VMEM((DT, 8, BBLK), jnp.float32) for _ in range(NBUF)]
            + [pltpu.SemaphoreType.DMA for _ in range(2 * NBUF)]
        ),
    )
    def gather(table_hbm, x_hbm, out_hbm, xblk, *bufs):
        icol = list(bufs[0:NBUF])
        rows = list(bufs[NBUF:2 * NBUF])
        trans = list(bufs[2 * NBUF:3 * NBUF])
        gsem = list(bufs[3 * NBUF:4 * NBUF])
        osem = list(bufs[4 * NBUF:5 * NBUF])

        wid = lax.axis_index("s") * NC + lax.axis_index("c")
        b_per_tile = BT_PER_TILE * BBLK
        bbase = wid * b_per_tile

        pltpu.sync_copy(x_hbm.at[pl.ds(bbase, b_per_tile)], xblk)

        def unit_btf(u):
            bt_local = u // FIELDS
            f = u - bt_local * FIELDS
            return bt_local, f

        def start_gather(u, p):
            bt_local, f = unit_btf(u)
            _extract_icol(xblk, icol[p], bt_local, f)
            pltpu.async_copy(table_hbm.at[icol[p]], rows[p], gsem[p])

        def finish_unit(u, p, drain_prev, start_next):
            bt_local, f = unit_btf(u)
            btg = wid * BT_PER_TILE + bt_local
            pltpu.make_async_copy(table_hbm.at[icol[p]], rows[p],
                                  gsem[p]).wait()
            if drain_prev:
                pltpu.make_async_copy(trans[p], out_hbm.at[f, :, btg],
                                      osem[p]).wait()
            _transpose_unit(rows[p], trans[p])
            pltpu.async_copy(trans[p], out_hbm.at[f, :, btg], osem[p])
            if start_next:
                start_gather(u + NBUF, p)

        for p in range(NBUF):
            start_gather(p, p)
        for p in range(NBUF):
            finish_unit(p, p, drain_prev=False, start_next=True)

        @pl.loop(NBUF, UNITS - NBUF, step=NBUF)
        def _(u):
            for p in range(NBUF):
                finish_unit(u + p, p, drain_prev=True, start_next=True)

        for p in range(NBUF):
            finish_unit(UNITS - NBUF + p, p, drain_prev=True,
                        start_next=False)

        last_bt = BT_PER_TILE - 1
        btg = wid * BT_PER_TILE + last_bt
        for p in range(NBUF):
            pltpu.make_async_copy(trans[p],
                                  out_hbm.at[FIELDS - NBUF + p, :, btg],
                                  osem[p]).wait()

    return gather


def kernel(x, table):
    out5 = _build_gather()(table, x)
    out = out5.transpose(2, 4, 0, 1, 3).reshape(BATCH, FIELDS, DIM)
    return out
